# Initial kernel scaffold; baseline (speedup 1.0000x reference)
import jax
import jax.numpy as jnp
from jax.experimental import pallas as pl


def kernel(node_feats, edge_feats, pos_enc, edge_index, graph_ids, params):
    raise NotImplementedError("write your pallas kernel here")


# speedup vs baseline: 12.3694x; 12.3694x over previous
import functools

import jax
import jax.numpy as jnp
from jax import lax
from jax.experimental import pallas as pl
from jax.experimental.pallas import tpu as pltpu
from jax.experimental.pallas import tpu_sc as plsc

F32 = jnp.float32
_EPS_BN = 1e-5
_EPS_LN = 1e-5
_EPS_AGG = 1e-6
_NHEADS = 8

_SC_CORES = 2
_SC_SUBCORES = 16
_NW = _SC_CORES * _SC_SUBCORES


def _pick_chunk(n, cap=128):
    for c in range(min(cap, n), 0, -1):
        if c % 8 == 0 and n % c == 0:
            return c
    return None


def _sc_mesh():
    return plsc.VectorSubcoreMesh(
        core_axis_name="c", subcore_axis_name="s",
        num_cores=_SC_CORES, num_subcores=_SC_SUBCORES)


def _sc_gather(table, idx):
    n_rows, d = table.shape
    e = idx.shape[0]
    per_w = e // _NW
    assert per_w * _NW == e
    c = _pick_chunk(per_w)
    iters = per_w // c

    @functools.partial(
        pl.kernel, mesh=_sc_mesh(),
        out_type=jax.ShapeDtypeStruct((e, d), F32),
        scratch_types=[
            pltpu.VMEM((c,), jnp.int32),
            pltpu.VMEM((c, d), F32),
            pltpu.SemaphoreType.DMA,
        ])
    def k(table_hbm, idx_hbm, out_hbm, idx_v, rows_v, sem):
        wid = lax.axis_index("s") * _SC_CORES + lax.axis_index("c")
        base = wid * per_w

        def body(i, carry):
            off = base + i * c
            pltpu.sync_copy(idx_hbm.at[pl.ds(off, c)], idx_v)
            pltpu.async_copy(table_hbm.at[idx_v], rows_v, sem).wait()
            pltpu.sync_copy(rows_v, out_hbm.at[pl.ds(off, c)])
            return carry

        lax.fori_loop(0, iters, body, 0)

    return k(table, idx)


def _sc_scatter_add(values, idx, n_out_pad):
    e, d = values.shape
    assert n_out_pad % (_SC_SUBCORES * 8) == 0
    per_core = e // _SC_CORES
    per_t = per_core // _SC_SUBCORES
    assert per_t * _NW == e
    c = _pick_chunk(per_t)
    iters = per_t // c
    rows_pt = n_out_pad // _SC_SUBCORES
    zc = _pick_chunk(rows_pt, cap=125) or rows_pt
    assert rows_pt % zc == 0 and zc % 8 == 0
    zits = rows_pt // zc

    @functools.partial(
        pl.kernel, mesh=_sc_mesh(),
        out_type=jax.ShapeDtypeStruct((_SC_CORES, n_out_pad, d), F32),
        scratch_types=[
            pltpu.VMEM((c,), jnp.int32),
            pltpu.VMEM((c, d), F32),
            pltpu.VMEM((zc, d), F32),
            pltpu.VMEM_SHARED((n_out_pad, d), F32),
        ])
    def k(vals_hbm, idx_hbm, out_hbm, idx_v, vals_v, zbuf, acc):
        cid = lax.axis_index("c")
        sid = lax.axis_index("s")

        def zrow(r, carry):
            for kk in range(d // 16):
                zbuf[r, pl.ds(kk * 16, 16)] = jnp.zeros((16,), F32)
            return carry
        lax.fori_loop(0, zc, zrow, 0)

        def zcopy(j, carry):
            pltpu.sync_copy(zbuf, acc.at[pl.ds(sid * rows_pt + j * zc, zc)])
            return carry
        lax.fori_loop(0, zits, zcopy, 0)
        plsc.subcore_barrier()

        base = cid * per_core + sid * per_t

        def body(i, carry):
            off = base + i * c
            pltpu.sync_copy(idx_hbm.at[pl.ds(off, c)], idx_v)
            pltpu.sync_copy(vals_hbm.at[pl.ds(off, c)], vals_v)
            pltpu.sync_copy(vals_v, acc.at[idx_v], add=True)
            return carry
        lax.fori_loop(0, iters, body, 0)
        plsc.subcore_barrier()

        def ocopy(j, carry):
            r0 = sid * rows_pt + j * zc
            pltpu.sync_copy(acc.at[pl.ds(r0, zc)], zbuf)
            pltpu.sync_copy(zbuf, out_hbm.at[cid, pl.ds(r0, zc)])
            return carry
        lax.fori_loop(0, zits, ocopy, 0)

    return k(values, idx)


def _pick_block(r):
    for b in (512, 1024, 1000, 256, 128, 200, 40, 16, 8):
        if r % b == 0:
            return b
    return r


def _mm(x, w, b, act=None, ln=None):
    r, kdim = x.shape
    n = w.shape[1]
    br = _pick_block(r)
    grid = (r // br,)
    ins = [x, w, b.reshape(1, n)]
    in_specs = [
        pl.BlockSpec((br, kdim), lambda i: (i, 0)),
        pl.BlockSpec((kdim, n), lambda i: (0, 0)),
        pl.BlockSpec((1, n), lambda i: (0, 0)),
    ]
    if ln is not None:
        g, beta = ln
        ins += [g.reshape(1, n), beta.reshape(1, n)]
        in_specs += [pl.BlockSpec((1, n), lambda i: (0, 0))] * 2

    def body(x_ref, w_ref, b_ref, *rest):
        out_ref = rest[-1]
        y = jnp.dot(x_ref[...], w_ref[...], preferred_element_type=F32)
        y = y + b_ref[...]
        if act == "relu":
            y = jnp.maximum(y, 0.0)
        if ln is not None:
            g_ref, beta_ref = rest[0], rest[1]
            m = jnp.mean(y, axis=-1, keepdims=True)
            v = jnp.mean((y - m) ** 2, axis=-1, keepdims=True)
            y = (y - m) / jnp.sqrt(v + _EPS_LN) * g_ref[...] + beta_ref[...]
        out_ref[...] = y

    return pl.pallas_call(
        body, grid=grid, in_specs=in_specs,
        out_specs=pl.BlockSpec((br, n), lambda i: (i, 0)),
        out_shape=jax.ShapeDtypeStruct((r, n), F32),
    )(*ins)


def _edge_gcn(e_arr, gdst, gsrc, b3w, b3b):
    e, d = e_arr.shape
    br = _pick_block(e)
    grid = (e // br,)

    def body(e_ref, gd_ref, gb2_ref, ga2_ref, gc2_ref, w_ref, b_ref,
             eh_ref, sg_ref, sa_ref, sc_ref, st_ref):
        b3 = jnp.dot(e_ref[...], w_ref[...], preferred_element_type=F32)
        eh = gd_ref[...] + gb2_ref[...] + b3 + b_ref[...]
        sg = jax.nn.sigmoid(eh)
        eh_ref[...] = eh
        sg_ref[...] = sg
        sa_ref[...] = sg * ga2_ref[...]
        sc_ref[...] = sg * gc2_ref[...]

        @pl.when(pl.program_id(0) == 0)
        def _():
            st_ref[...] = jnp.zeros_like(st_ref)

        s1 = jnp.sum(eh, axis=0, keepdims=True)
        s2 = jnp.sum(eh * eh, axis=0, keepdims=True)
        st_ref[...] += jnp.concatenate(
            [s1, s2, jnp.zeros((6, d), F32)], axis=0)

    shp = jax.ShapeDtypeStruct((e, d), F32)
    return pl.pallas_call(
        body, grid=grid,
        in_specs=[
            pl.BlockSpec((br, d), lambda i: (i, 0)),
            pl.BlockSpec((br, d), lambda i: (i, 0)),
            pl.BlockSpec((br, d), lambda i: (i, 0)),
            pl.BlockSpec((br, d), lambda i: (i, 1)),
            pl.BlockSpec((br, d), lambda i: (i, 2)),
            pl.BlockSpec((d, d), lambda i: (0, 0)),
            pl.BlockSpec((1, d), lambda i: (0, 0)),
        ],
        out_specs=[
            pl.BlockSpec((br, d), lambda i: (i, 0)),
            pl.BlockSpec((br, d), lambda i: (i, 0)),
            pl.BlockSpec((br, d), lambda i: (i, 0)),
            pl.BlockSpec((br, d), lambda i: (i, 0)),
            pl.BlockSpec((8, d), lambda i: (0, 0)),
        ],
        out_shape=[shp, shp, shp, shp,
                   jax.ShapeDtypeStruct((8, d), F32)],
    )(e_arr, gdst, gsrc, gsrc, gsrc, b3w, b3b.reshape(1, d))


def _bn_apply(x, res, stats, g, b, count):
    r, d = x.shape
    br = _pick_block(r)

    def body(x_ref, res_ref, st_ref, g_ref, b_ref, o_ref):
        m = st_ref[0:1, :] / count
        v = st_ref[1:2, :] / count - m * m
        y = (x_ref[...] - m) / jnp.sqrt(v + _EPS_BN) * g_ref[...] + b_ref[...]
        o_ref[...] = jnp.maximum(y, 0.0) + res_ref[...]

    return pl.pallas_call(
        body, grid=(r // br,),
        in_specs=[
            pl.BlockSpec((br, d), lambda i: (i, 0)),
            pl.BlockSpec((br, d), lambda i: (i, 0)),
            pl.BlockSpec((8, d), lambda i: (0, 0)),
            pl.BlockSpec((1, d), lambda i: (0, 0)),
            pl.BlockSpec((1, d), lambda i: (0, 0)),
        ],
        out_specs=pl.BlockSpec((br, d), lambda i: (i, 0)),
        out_shape=jax.ShapeDtypeStruct((r, d), F32),
    )(x, res, stats, g.reshape(1, d), b.reshape(1, d))


def _node_gcn_a(a1h, c1p, ssig, snh, snp, p_in, p_i):
    n, d = a1h.shape
    br = _pick_block(n)

    def body(a1_ref, c1_ref, ss_ref, nh_ref, np_ref, pin_ref, pi_ref,
             h_ref, p_ref, st_ref):
        den = ss_ref[0] + ss_ref[1] + _EPS_AGG
        h_raw = a1_ref[...] + (nh_ref[0] + nh_ref[1]) / den
        p_out = jnp.tanh(c1_ref[...] + (np_ref[0] + np_ref[1]) / den)
        h_ref[...] = h_raw
        p_ref[...] = p_out + pin_ref[...] + pi_ref[...]

        @pl.when(pl.program_id(0) == 0)
        def _():
            st_ref[...] = jnp.zeros_like(st_ref)

        s1 = jnp.sum(h_raw, axis=0, keepdims=True)
        s2 = jnp.sum(h_raw * h_raw, axis=0, keepdims=True)
        st_ref[...] += jnp.concatenate(
            [s1, s2, jnp.zeros((6, d), F32)], axis=0)

    part = pl.BlockSpec((2, br, d), lambda i: (0, i, 0))
    blk = pl.BlockSpec((br, d), lambda i: (i, 0))
    shp = jax.ShapeDtypeStruct((n, d), F32)
    return pl.pallas_call(
        body, grid=(n // br,),
        in_specs=[blk, blk, part, part, part, blk, blk],
        out_specs=[blk, blk, pl.BlockSpec((8, d), lambda i: (0, 0))],
        out_shape=[shp, shp, jax.ShapeDtypeStruct((8, d), F32)],
    )(a1h, c1p, ssig, snh, snp, p_in, p_i)


def _gt_edge(e_arr, gq, gk, gv, ew, eb):
    e, d = e_arr.shape
    br = _pick_block(e)
    dh = d // _NHEADS

    def body(e_ref, gq_ref, gk_ref, gv_ref, w_ref, b_ref,
             att_ref, wb_ref, wv_ref):
        ee = jnp.dot(e_ref[...], w_ref[...], preferred_element_type=F32)
        ee = ee + b_ref[...]
        score = gq_ref[...] * gk_ref[...] * (1.0 / (dh ** 0.5)) * ee
        att_ref[...] = score
        rr = lax.broadcasted_iota(jnp.int32, (d, d), 0) // dh
        cc = lax.broadcasted_iota(jnp.int32, (d, d), 1) // dh
        mask = (rr == cc).astype(F32)
        s = jnp.dot(score, mask, preferred_element_type=F32)
        wb = jnp.exp(jnp.clip(s, -5.0, 5.0))
        wb_ref[...] = wb
        wv_ref[...] = wb * gv_ref[...]

    blk = pl.BlockSpec((br, d), lambda i: (i, 0))
    shp = jax.ShapeDtypeStruct((e, d), F32)
    return pl.pallas_call(
        body, grid=(e // br,),
        in_specs=[
            blk,
            pl.BlockSpec((br, d), lambda i: (i, 1)),
            pl.BlockSpec((br, d), lambda i: (i, 3)),
            pl.BlockSpec((br, d), lambda i: (i, 4)),
            pl.BlockSpec((d, d), lambda i: (0, 0)),
            pl.BlockSpec((1, d), lambda i: (0, 0)),
        ],
        out_specs=[blk, blk, blk],
        out_shape=[shp, shp, shp],
    )(e_arr, gq, gk, gv, ew, eb.reshape(1, d))


def _gt_post(att_num, att_den, x_in, ow, ob, ln1, ff1, ff2, ln2,
             x1, x_i, w1v, w2v):
    r, d = x_in.shape
    br = _pick_block(r)
    dff = ff1[0].shape[1]
    has_den = att_den is not None

    def body(*refs):
        i = 0
        num_ref = refs[i]; i += 1
        if has_den:
            den_ref = refs[i]; i += 1
        (xin_ref, ow_ref, ob_ref, g1_ref, b1_ref, f1w_ref, f1b_ref,
         f2w_ref, f2b_ref, g2_ref, b2_ref, x1_ref, xi_ref, w1_ref,
         w2_ref, out_ref) = refs[i:]
        if has_den:
            att = (num_ref[0] + num_ref[1]) / (den_ref[0] + den_ref[1]
                                               + _EPS_AGG)
        else:
            att = num_ref[...]
        y = jnp.dot(att, ow_ref[...], preferred_element_type=F32)
        y = y + ob_ref[...] + xin_ref[...]
        m = jnp.mean(y, axis=-1, keepdims=True)
        v = jnp.mean((y - m) ** 2, axis=-1, keepdims=True)
        y = (y - m) / jnp.sqrt(v + _EPS_LN) * g1_ref[...] + b1_ref[...]
        z = jnp.dot(y, f1w_ref[...], preferred_element_type=F32)
        z = jnp.maximum(z + f1b_ref[...], 0.0)
        z = jnp.dot(z, f2w_ref[...], preferred_element_type=F32)
        z = z + f2b_ref[...] + y
        m = jnp.mean(z, axis=-1, keepdims=True)
        v = jnp.mean((z - m) ** 2, axis=-1, keepdims=True)
        x2 = (z - m) / jnp.sqrt(v + _EPS_LN) * g2_ref[...] + b2_ref[...]
        out_ref[...] = x1_ref[...] * w1_ref[...] + x2 * w2_ref[...] \
            + xi_ref[...]

    blk = pl.BlockSpec((br, d), lambda i: (i, 0))
    row = pl.BlockSpec((1, d), lambda i: (0, 0))
    part = pl.BlockSpec((2, br, d), lambda i: (0, i, 0))
    ins = []
    in_specs = []
    if has_den:
        ins += [att_num, att_den]
        in_specs += [part, part]
    else:
        ins += [att_num]
        in_specs += [blk]
    ins += [x_in, ow, ob.reshape(1, d), ln1[0].reshape(1, d),
            ln1[1].reshape(1, d), ff1[0], ff1[1].reshape(1, dff),
            ff2[0], ff2[1].reshape(1, d), ln2[0].reshape(1, d),
            ln2[1].reshape(1, d), x1, x_i, w1v, w2v]
    in_specs += [blk, pl.BlockSpec((d, d), lambda i: (0, 0)), row, row, row,
                 pl.BlockSpec((d, dff), lambda i: (0, 0)),
                 pl.BlockSpec((1, dff), lambda i: (0, 0)),
                 pl.BlockSpec((dff, d), lambda i: (0, 0)), row, row, row,
                 blk, blk, row, row]
    return pl.pallas_call(
        body, grid=(r // br,), in_specs=in_specs, out_specs=blk,
        out_shape=jax.ShapeDtypeStruct((r, d), F32),
    )(*ins)


def _head(hg_parts, n_graphs, bn_g, bn_b, w1, b1, w2, b2):
    p, d = hg_parts.shape[1], hg_parts.shape[2]

    def body(hp_ref, g_ref, b_ref, w1_ref, b1_ref, w2_ref, b2_ref, o_ref):
        hg = hp_ref[0, :n_graphs, :] + hp_ref[1, :n_graphs, :]
        m = jnp.mean(hg, axis=0, keepdims=True)
        v = jnp.mean((hg - m) ** 2, axis=0, keepdims=True)
        x = (hg - m) / jnp.sqrt(v + _EPS_BN) * g_ref[...] + b_ref[...]
        x = jax.nn.gelu(x)
        x = jnp.dot(x, w1_ref[...], preferred_element_type=F32) + b1_ref[...]
        x = jnp.dot(x, w2_ref[...], preferred_element_type=F32) + b2_ref[...]
        o_ref[...] = x

    return pl.pallas_call(
        body,
        in_specs=[
            pl.BlockSpec((2, p, d), lambda: (0, 0, 0)),
            pl.BlockSpec((1, d), lambda: (0, 0)),
            pl.BlockSpec((1, d), lambda: (0, 0)),
            pl.BlockSpec((d, d), lambda: (0, 0)),
            pl.BlockSpec((1, d), lambda: (0, 0)),
            pl.BlockSpec((d, 1), lambda: (0, 0)),
            pl.BlockSpec((1, 1), lambda: (0, 0)),
        ],
        out_specs=pl.BlockSpec((n_graphs, 1), lambda: (0, 0)),
        out_shape=jax.ShapeDtypeStruct((n_graphs, 1), F32),
    )(hg_parts, bn_g.reshape(1, d), bn_b.reshape(1, d), w1,
      b1.reshape(1, d), w2, b2.reshape(1, 1))


def kernel(node_feats, edge_feats, pos_enc, edge_index, graph_ids, params):
    n = node_feats.shape[0]
    e = edge_feats.shape[0]
    d = params["node_enc_w"].shape[1]
    src = edge_index[0].astype(jnp.int32)
    dst = edge_index[1].astype(jnp.int32)

    h = _mm(node_feats, params["node_enc_w"], params["node_enc_b"],
            ln=(params["node_ln_g"], params["node_ln_b"]))
    e_arr = _mm(edge_feats, params["edge_enc_w"], params["edge_enc_b"],
                ln=(params["edge_ln_g"], params["edge_ln_b"]))
    p = _mm(pos_enc, params["pose_enc_w"], params["pose_enc_b"])
    h_i, e_i, p_i = h, e_arr, p

    n_acc = ((n + 1279) // 1280) * 1280
    w1v = jnp.broadcast_to(params["weight1"], (1, d)).astype(F32)
    w2v = jnp.broadcast_to(params["weight2"], (1, d)).astype(F32)
    zcol = jnp.zeros((d, d), F32)

    for lg, lt in zip(params["gcn"], params["gt"]):
        hp = jnp.concatenate([h, p], axis=-1)
        w_src = jnp.concatenate([
            jnp.concatenate([lg["B2_w"], zcol], 0),
            lg["A2_w"],
            jnp.concatenate([zcol, lg["C2_w"]], 0),
            jnp.concatenate([lt["K_w"], zcol], 0),
            jnp.concatenate([lt["V_w"], zcol], 0)], 1)
        b_src = jnp.concatenate([lg["B2_b"], lg["A2_b"], lg["C2_b"],
                                 lt["K_b"], lt["V_b"]])
        t_src = _mm(hp, w_src, b_src)
        w_dst = jnp.concatenate([
            jnp.concatenate([lg["B1_w"], zcol], 0),
            jnp.concatenate([lt["Q_w"], zcol], 0)], 1)
        b_dst = jnp.concatenate([lg["B1_b"], lt["Q_b"]])
        t_dst = _mm(hp, w_dst, b_dst)
        w_a1c1 = jnp.concatenate([
            lg["A1_w"], jnp.concatenate([zcol, lg["C1_w"]], 0)], 1)
        b_a1c1 = jnp.concatenate([lg["A1_b"], lg["C1_b"]])
        t_a1c1 = _mm(hp, w_a1c1, b_a1c1)
        a1h = t_a1c1[:, :d]
        c1p = t_a1c1[:, d:]

        gsrc = _sc_gather(t_src, src)
        gdst = _sc_gather(t_dst, dst)

        e_hat, sigma, sa, sc_v, e_stats = _edge_gcn(
            e_arr, gdst, gsrc, lg["B3_w"], lg["B3_b"])

        ssig = _sc_scatter_add(sigma, dst, n_acc)
        snh = _sc_scatter_add(sa, dst, n_acc)
        snp = _sc_scatter_add(sc_v, dst, n_acc)

        h_raw, p_new, h_stats = _node_gcn_a(a1h, c1p, ssig, snh, snp, p, p_i)
        h1 = _bn_apply(h_raw, h, h_stats, lg["bn_h_g"], lg["bn_h_b"],
                       float(n))
        e1 = _bn_apply(e_hat, e_arr, e_stats, lg["bn_e_g"], lg["bn_e_b"],
                       float(e))

        e_att, wb, wv = _gt_edge(e_arr, gdst, gsrc, gsrc,
                                 lt["E_w"], lt["E_b"])
        swv = _sc_scatter_add(wv, dst, n_acc)
        swb = _sc_scatter_add(wb, dst, n_acc)

        h = _gt_post(swv, swb, h, lt["Oh_w"], lt["Oh_b"],
                     (lt["ln1_h_g"], lt["ln1_h_b"]),
                     (lt["ffh_w1"], lt["ffh_b1"]),
                     (lt["ffh_w2"], lt["ffh_b2"]),
                     (lt["ln2_h_g"], lt["ln2_h_b"]),
                     h1, h_i, w1v, w2v)
        e_arr = _gt_post(e_att, None, e_arr, lt["Oe_w"], lt["Oe_b"],
                         (lt["ln1_e_g"], lt["ln1_e_b"]),
                         (lt["ffe_w1"], lt["ffe_b1"]),
                         (lt["ffe_w2"], lt["ffe_b2"]),
                         (lt["ln2_e_g"], lt["ln2_e_b"]),
                         e1, e_i, w1v, w2v)
        p = p_new

    n_graphs = 64
    pool_pad_rows = 128
    n_pad = ((n + _NW * 8 - 1) // (_NW * 8)) * (_NW * 8)
    extra = n_pad - n
    h_pool = h if extra == 0 else jnp.concatenate(
        [h, jnp.zeros((extra, d), F32)], axis=0)
    gid = graph_ids.astype(jnp.int32)
    gid_pad = gid if extra == 0 else jnp.concatenate(
        [gid, jnp.full((extra,), n_graphs, jnp.int32)])
    hg_parts = _sc_scatter_add(h_pool, gid_pad, pool_pad_rows)

    return _head(hg_parts, n_graphs, params["mlp_bn_g"], params["mlp_bn_b"],
                 params["mlp_w1"], params["mlp_b1"],
                 params["mlp_w2"], params["mlp_b2"])


# speedup vs baseline: 12.6512x; 1.0228x over previous
import functools

import jax
import jax.numpy as jnp
from jax import lax
from jax.experimental import pallas as pl
from jax.experimental.pallas import tpu as pltpu
from jax.experimental.pallas import tpu_sc as plsc

F32 = jnp.float32
_EPS_BN = 1e-5
_EPS_LN = 1e-5
_EPS_AGG = 1e-6
_NHEADS = 8

_SC_CORES = 2
_SC_SUBCORES = 16
_NW = _SC_CORES * _SC_SUBCORES


def _pick_chunk(n, cap=128):
    for c in range(min(cap, n), 0, -1):
        if c % 8 == 0 and n % c == 0:
            return c
    return None


def _pick_chunk128(n, cap):
    c = (min(cap, n) // 128) * 128
    while c >= 128:
        if n % c == 0:
            return c
        c -= 128
    raise ValueError(f"no 128-multiple chunk for {n} cap {cap}")


_ROWBUF_BYTES = 440 * 1024


def _sc_mesh():
    return plsc.VectorSubcoreMesh(
        core_axis_name="c", subcore_axis_name="s",
        num_cores=_SC_CORES, num_subcores=_SC_SUBCORES)


def _sc_gather(table, idx, cap=400):
    n_rows, d = table.shape
    e = idx.shape[0]
    per_w = e // _NW
    assert per_w * _NW == e
    c = _pick_chunk128(per_w, cap=min(cap, _ROWBUF_BYTES // (d * 4)))
    iters = per_w // c
    idx2 = idx.reshape(_NW, iters, c)

    @functools.partial(
        pl.kernel, mesh=_sc_mesh(),
        out_type=jax.ShapeDtypeStruct((e, d), F32),
        scratch_types=[
            pltpu.VMEM((iters, c), jnp.int32),
            pltpu.VMEM((c, d), F32),
            pltpu.SemaphoreType.DMA,
        ])
    def k(table_hbm, idx_hbm, out_hbm, idx_v, rows_v, sem):
        wid = lax.axis_index("s") * _SC_CORES + lax.axis_index("c")
        base = wid * per_w
        pltpu.sync_copy(idx_hbm.at[wid], idx_v)

        def body(i, carry):
            pltpu.async_copy(table_hbm.at[idx_v.at[i]], rows_v, sem).wait()
            pltpu.sync_copy(rows_v, out_hbm.at[pl.ds(base + i * c, c)])
            return carry

        lax.fori_loop(0, iters, body, 0)

    return k(table, idx2)


def _sc_scatter_add(values, idx, n_out_pad):
    e, d = values.shape
    assert n_out_pad % (_SC_SUBCORES * 8) == 0
    per_core = e // _SC_CORES
    per_t = per_core // _SC_SUBCORES
    assert per_t * _NW == e
    c = _pick_chunk128(per_t, cap=min(400, _ROWBUF_BYTES // (d * 4)))
    iters = per_t // c
    idx2 = idx.reshape(_NW, iters, c)
    rows_pt = n_out_pad // _SC_SUBCORES
    zc = _pick_chunk(rows_pt, cap=125) or rows_pt
    assert rows_pt % zc == 0 and zc % 8 == 0
    zits = rows_pt // zc

    @functools.partial(
        pl.kernel, mesh=_sc_mesh(),
        out_type=jax.ShapeDtypeStruct((_SC_CORES, n_out_pad, d), F32),
        scratch_types=[
            pltpu.VMEM((iters, c), jnp.int32),
            pltpu.VMEM((c, d), F32),
            pltpu.VMEM((zc, d), F32),
            pltpu.VMEM_SHARED((n_out_pad, d), F32),
        ])
    def k(vals_hbm, idx_hbm, out_hbm, idx_v, vals_v, zbuf, acc):
        cid = lax.axis_index("c")
        sid = lax.axis_index("s")
        wid = cid * _SC_SUBCORES + sid
        pltpu.sync_copy(idx_hbm.at[wid], idx_v)

        def zrow(r, carry):
            for kk in range(d // 16):
                zbuf[r, pl.ds(kk * 16, 16)] = jnp.zeros((16,), F32)
            return carry
        lax.fori_loop(0, zc, zrow, 0)

        def zcopy(j, carry):
            pltpu.sync_copy(zbuf, acc.at[pl.ds(sid * rows_pt + j * zc, zc)])
            return carry
        lax.fori_loop(0, zits, zcopy, 0)
        plsc.subcore_barrier()

        base = cid * per_core + sid * per_t

        def body(i, carry):
            off = base + i * c
            pltpu.sync_copy(vals_hbm.at[pl.ds(off, c)], vals_v)
            pltpu.sync_copy(vals_v, acc.at[idx_v.at[i]], add=True)
            return carry
        lax.fori_loop(0, iters, body, 0)
        plsc.subcore_barrier()

        def ocopy(j, carry):
            r0 = sid * rows_pt + j * zc
            pltpu.sync_copy(acc.at[pl.ds(r0, zc)], zbuf)
            pltpu.sync_copy(zbuf, out_hbm.at[cid, pl.ds(r0, zc)])
            return carry
        lax.fori_loop(0, zits, ocopy, 0)

    return k(values, idx2)


def _pick_block(r):
    for b in (512, 1024, 1000, 256, 128, 200, 40, 16, 8):
        if r % b == 0:
            return b
    return r


def _mm(x, w, b, act=None, ln=None):
    r, kdim = x.shape
    n = w.shape[1]
    br = _pick_block(r)
    grid = (r // br,)
    ins = [x, w, b.reshape(1, n)]
    in_specs = [
        pl.BlockSpec((br, kdim), lambda i: (i, 0)),
        pl.BlockSpec((kdim, n), lambda i: (0, 0)),
        pl.BlockSpec((1, n), lambda i: (0, 0)),
    ]
    if ln is not None:
        g, beta = ln
        ins += [g.reshape(1, n), beta.reshape(1, n)]
        in_specs += [pl.BlockSpec((1, n), lambda i: (0, 0))] * 2

    def body(x_ref, w_ref, b_ref, *rest):
        out_ref = rest[-1]
        y = jnp.dot(x_ref[...], w_ref[...], preferred_element_type=F32)
        y = y + b_ref[...]
        if act == "relu":
            y = jnp.maximum(y, 0.0)
        if ln is not None:
            g_ref, beta_ref = rest[0], rest[1]
            m = jnp.mean(y, axis=-1, keepdims=True)
            v = jnp.mean((y - m) ** 2, axis=-1, keepdims=True)
            y = (y - m) / jnp.sqrt(v + _EPS_LN) * g_ref[...] + beta_ref[...]
        out_ref[...] = y

    return pl.pallas_call(
        body, grid=grid, in_specs=in_specs,
        out_specs=pl.BlockSpec((br, n), lambda i: (i, 0)),
        out_shape=jax.ShapeDtypeStruct((r, n), F32),
    )(*ins)


def _edge_gcn(e_arr, gdst, gsrc, b3w, b3b, out_rows=None):
    e, d = e_arr.shape
    br = _pick_block(e)
    grid = (e // br,)
    out_rows = out_rows or e

    def body(e_ref, gd_ref, gb2_ref, ga2_ref, gc2_ref, w_ref, b_ref,
             eh_ref, sg_ref, sa_ref, sc_ref, st_ref):
        b3 = jnp.dot(e_ref[...], w_ref[...], preferred_element_type=F32)
        eh = gd_ref[...] + gb2_ref[...] + b3 + b_ref[...]
        sg = jax.nn.sigmoid(eh)
        eh_ref[...] = eh
        sg_ref[...] = sg
        sa_ref[...] = sg * ga2_ref[...]
        sc_ref[...] = sg * gc2_ref[...]

        @pl.when(pl.program_id(0) == 0)
        def _():
            st_ref[...] = jnp.zeros_like(st_ref)

        s1 = jnp.sum(eh, axis=0, keepdims=True)
        s2 = jnp.sum(eh * eh, axis=0, keepdims=True)
        st_ref[...] += jnp.concatenate(
            [s1, s2, jnp.zeros((6, d), F32)], axis=0)

    shp = jax.ShapeDtypeStruct((e, d), F32)
    shp_p = jax.ShapeDtypeStruct((out_rows, d), F32)
    return pl.pallas_call(
        body, grid=grid,
        in_specs=[
            pl.BlockSpec((br, d), lambda i: (i, 0)),
            pl.BlockSpec((br, d), lambda i: (i, 0)),
            pl.BlockSpec((br, d), lambda i: (i, 0)),
            pl.BlockSpec((br, d), lambda i: (i, 1)),
            pl.BlockSpec((br, d), lambda i: (i, 2)),
            pl.BlockSpec((d, d), lambda i: (0, 0)),
            pl.BlockSpec((1, d), lambda i: (0, 0)),
        ],
        out_specs=[
            pl.BlockSpec((br, d), lambda i: (i, 0)),
            pl.BlockSpec((br, d), lambda i: (i, 0)),
            pl.BlockSpec((br, d), lambda i: (i, 0)),
            pl.BlockSpec((br, d), lambda i: (i, 0)),
            pl.BlockSpec((8, d), lambda i: (0, 0)),
        ],
        out_shape=[shp, shp_p, shp_p, shp_p,
                   jax.ShapeDtypeStruct((8, d), F32)],
    )(e_arr, gdst, gsrc, gsrc, gsrc, b3w, b3b.reshape(1, d))


def _bn_apply(x, res, stats, g, b, count):
    r, d = x.shape
    br = _pick_block(r)

    def body(x_ref, res_ref, st_ref, g_ref, b_ref, o_ref):
        m = st_ref[0:1, :] / count
        v = st_ref[1:2, :] / count - m * m
        y = (x_ref[...] - m) / jnp.sqrt(v + _EPS_BN) * g_ref[...] + b_ref[...]
        o_ref[...] = jnp.maximum(y, 0.0) + res_ref[...]

    return pl.pallas_call(
        body, grid=(r // br,),
        in_specs=[
            pl.BlockSpec((br, d), lambda i: (i, 0)),
            pl.BlockSpec((br, d), lambda i: (i, 0)),
            pl.BlockSpec((8, d), lambda i: (0, 0)),
            pl.BlockSpec((1, d), lambda i: (0, 0)),
            pl.BlockSpec((1, d), lambda i: (0, 0)),
        ],
        out_specs=pl.BlockSpec((br, d), lambda i: (i, 0)),
        out_shape=jax.ShapeDtypeStruct((r, d), F32),
    )(x, res, stats, g.reshape(1, d), b.reshape(1, d))


def _node_gcn_a(a1h, c1p, ssig, snh, snp, p_in, p_i):
    n, d = a1h.shape
    br = _pick_block(n)

    def body(a1_ref, c1_ref, ss_ref, nh_ref, np_ref, pin_ref, pi_ref,
             h_ref, p_ref, st_ref):
        den = ss_ref[0] + ss_ref[1] + _EPS_AGG
        h_raw = a1_ref[...] + (nh_ref[0] + nh_ref[1]) / den
        p_out = jnp.tanh(c1_ref[...] + (np_ref[0] + np_ref[1]) / den)
        h_ref[...] = h_raw
        p_ref[...] = p_out + pin_ref[...] + pi_ref[...]

        @pl.when(pl.program_id(0) == 0)
        def _():
            st_ref[...] = jnp.zeros_like(st_ref)

        s1 = jnp.sum(h_raw, axis=0, keepdims=True)
        s2 = jnp.sum(h_raw * h_raw, axis=0, keepdims=True)
        st_ref[...] += jnp.concatenate(
            [s1, s2, jnp.zeros((6, d), F32)], axis=0)

    part = pl.BlockSpec((2, br, d), lambda i: (0, i, 0))
    blk = pl.BlockSpec((br, d), lambda i: (i, 0))
    shp = jax.ShapeDtypeStruct((n, d), F32)
    return pl.pallas_call(
        body, grid=(n // br,),
        in_specs=[blk, blk, part, part, part, blk, blk],
        out_specs=[blk, blk, pl.BlockSpec((8, d), lambda i: (0, 0))],
        out_shape=[shp, shp, jax.ShapeDtypeStruct((8, d), F32)],
    )(a1h, c1p, ssig, snh, snp, p_in, p_i)


def _gt_edge(e_arr, gq, gk, gv, ew, eb, out_rows=None):
    e, d = e_arr.shape
    br = _pick_block(e)
    dh = d // _NHEADS
    out_rows = out_rows or e

    def body(e_ref, gq_ref, gk_ref, gv_ref, w_ref, b_ref,
             att_ref, wb_ref, wv_ref):
        ee = jnp.dot(e_ref[...], w_ref[...], preferred_element_type=F32)
        ee = ee + b_ref[...]
        score = gq_ref[...] * gk_ref[...] * (1.0 / (dh ** 0.5)) * ee
        att_ref[...] = score
        rr = lax.broadcasted_iota(jnp.int32, (d, d), 0) // dh
        cc = lax.broadcasted_iota(jnp.int32, (d, d), 1) // dh
        mask = (rr == cc).astype(F32)
        s = jnp.dot(score, mask, preferred_element_type=F32)
        wb = jnp.exp(jnp.clip(s, -5.0, 5.0))
        wb_ref[...] = wb
        wv_ref[...] = wb * gv_ref[...]

    blk = pl.BlockSpec((br, d), lambda i: (i, 0))
    shp = jax.ShapeDtypeStruct((e, d), F32)
    shp_p = jax.ShapeDtypeStruct((out_rows, d), F32)
    return pl.pallas_call(
        body, grid=(e // br,),
        in_specs=[
            blk,
            pl.BlockSpec((br, d), lambda i: (i, 1)),
            pl.BlockSpec((br, d), lambda i: (i, 3)),
            pl.BlockSpec((br, d), lambda i: (i, 4)),
            pl.BlockSpec((d, d), lambda i: (0, 0)),
            pl.BlockSpec((1, d), lambda i: (0, 0)),
        ],
        out_specs=[blk, blk, blk],
        out_shape=[shp, shp_p, shp_p],
    )(e_arr, gq, gk, gv, ew, eb.reshape(1, d))


def _gt_post(att_num, att_den, x_in, ow, ob, ln1, ff1, ff2, ln2,
             x1, x_i, w1v, w2v):
    r, d = x_in.shape
    br = _pick_block(r)
    dff = ff1[0].shape[1]
    has_den = att_den is not None

    def body(*refs):
        i = 0
        num_ref = refs[i]; i += 1
        if has_den:
            den_ref = refs[i]; i += 1
        (xin_ref, ow_ref, ob_ref, g1_ref, b1_ref, f1w_ref, f1b_ref,
         f2w_ref, f2b_ref, g2_ref, b2_ref, x1_ref, xi_ref, w1_ref,
         w2_ref, out_ref) = refs[i:]
        if has_den:
            att = (num_ref[0] + num_ref[1]) / (den_ref[0] + den_ref[1]
                                               + _EPS_AGG)
        else:
            att = num_ref[...]
        y = jnp.dot(att, ow_ref[...], preferred_element_type=F32)
        y = y + ob_ref[...] + xin_ref[...]
        m = jnp.mean(y, axis=-1, keepdims=True)
        v = jnp.mean((y - m) ** 2, axis=-1, keepdims=True)
        y = (y - m) / jnp.sqrt(v + _EPS_LN) * g1_ref[...] + b1_ref[...]
        z = jnp.dot(y, f1w_ref[...], preferred_element_type=F32)
        z = jnp.maximum(z + f1b_ref[...], 0.0)
        z = jnp.dot(z, f2w_ref[...], preferred_element_type=F32)
        z = z + f2b_ref[...] + y
        m = jnp.mean(z, axis=-1, keepdims=True)
        v = jnp.mean((z - m) ** 2, axis=-1, keepdims=True)
        x2 = (z - m) / jnp.sqrt(v + _EPS_LN) * g2_ref[...] + b2_ref[...]
        out_ref[...] = x1_ref[...] * w1_ref[...] + x2 * w2_ref[...] \
            + xi_ref[...]

    blk = pl.BlockSpec((br, d), lambda i: (i, 0))
    row = pl.BlockSpec((1, d), lambda i: (0, 0))
    part = pl.BlockSpec((2, br, d), lambda i: (0, i, 0))
    ins = []
    in_specs = []
    if has_den:
        ins += [att_num, att_den]
        in_specs += [part, part]
    else:
        ins += [att_num]
        in_specs += [blk]
    ins += [x_in, ow, ob.reshape(1, d), ln1[0].reshape(1, d),
            ln1[1].reshape(1, d), ff1[0], ff1[1].reshape(1, dff),
            ff2[0], ff2[1].reshape(1, d), ln2[0].reshape(1, d),
            ln2[1].reshape(1, d), x1, x_i, w1v, w2v]
    in_specs += [blk, pl.BlockSpec((d, d), lambda i: (0, 0)), row, row, row,
                 pl.BlockSpec((d, dff), lambda i: (0, 0)),
                 pl.BlockSpec((1, dff), lambda i: (0, 0)),
                 pl.BlockSpec((dff, d), lambda i: (0, 0)), row, row, row,
                 blk, blk, row, row]
    return pl.pallas_call(
        body, grid=(r // br,), in_specs=in_specs, out_specs=blk,
        out_shape=jax.ShapeDtypeStruct((r, d), F32),
    )(*ins)


def _head(hg_parts, n_graphs, bn_g, bn_b, w1, b1, w2, b2):
    p, d = hg_parts.shape[1], hg_parts.shape[2]

    def body(hp_ref, g_ref, b_ref, w1_ref, b1_ref, w2_ref, b2_ref, o_ref):
        hg = hp_ref[0, :n_graphs, :] + hp_ref[1, :n_graphs, :]
        m = jnp.mean(hg, axis=0, keepdims=True)
        v = jnp.mean((hg - m) ** 2, axis=0, keepdims=True)
        x = (hg - m) / jnp.sqrt(v + _EPS_BN) * g_ref[...] + b_ref[...]
        x = jax.nn.gelu(x)
        x = jnp.dot(x, w1_ref[...], preferred_element_type=F32) + b1_ref[...]
        x = jnp.dot(x, w2_ref[...], preferred_element_type=F32) + b2_ref[...]
        o_ref[...] = x

    return pl.pallas_call(
        body,
        in_specs=[
            pl.BlockSpec((2, p, d), lambda: (0, 0, 0)),
            pl.BlockSpec((1, d), lambda: (0, 0)),
            pl.BlockSpec((1, d), lambda: (0, 0)),
            pl.BlockSpec((d, d), lambda: (0, 0)),
            pl.BlockSpec((1, d), lambda: (0, 0)),
            pl.BlockSpec((d, 1), lambda: (0, 0)),
            pl.BlockSpec((1, 1), lambda: (0, 0)),
        ],
        out_specs=pl.BlockSpec((n_graphs, 1), lambda: (0, 0)),
        out_shape=jax.ShapeDtypeStruct((n_graphs, 1), F32),
    )(hg_parts, bn_g.reshape(1, d), bn_b.reshape(1, d), w1,
      b1.reshape(1, d), w2, b2.reshape(1, 1))


def kernel(node_feats, edge_feats, pos_enc, edge_index, graph_ids, params):
    n = node_feats.shape[0]
    e = edge_feats.shape[0]
    d = params["node_enc_w"].shape[1]
    src = edge_index[0].astype(jnp.int32)
    dst = edge_index[1].astype(jnp.int32)
    e_pad = ((e + _NW * 128 - 1) // (_NW * 128)) * (_NW * 128)
    ext = e_pad - e
    src_g = jnp.concatenate([src, jnp.zeros((ext,), jnp.int32)])
    dst_g = jnp.concatenate([dst, jnp.zeros((ext,), jnp.int32)])
    dst_s = jnp.concatenate([dst, jnp.full((ext,), n, jnp.int32)])

    h = _mm(node_feats, params["node_enc_w"], params["node_enc_b"],
            ln=(params["node_ln_g"], params["node_ln_b"]))
    e_arr = _mm(edge_feats, params["edge_enc_w"], params["edge_enc_b"],
                ln=(params["edge_ln_g"], params["edge_ln_b"]))
    p = _mm(pos_enc, params["pose_enc_w"], params["pose_enc_b"])
    h_i, e_i, p_i = h, e_arr, p

    n_acc = ((n + 1279) // 1280) * 1280
    w1v = jnp.broadcast_to(params["weight1"], (1, d)).astype(F32)
    w2v = jnp.broadcast_to(params["weight2"], (1, d)).astype(F32)
    zcol = jnp.zeros((d, d), F32)

    for lg, lt in zip(params["gcn"], params["gt"]):
        hp = jnp.concatenate([h, p], axis=-1)
        w_src = jnp.concatenate([
            jnp.concatenate([lg["B2_w"], zcol], 0),
            lg["A2_w"],
            jnp.concatenate([zcol, lg["C2_w"]], 0),
            jnp.concatenate([lt["K_w"], zcol], 0),
            jnp.concatenate([lt["V_w"], zcol], 0)], 1)
        b_src = jnp.concatenate([lg["B2_b"], lg["A2_b"], lg["C2_b"],
                                 lt["K_b"], lt["V_b"]])
        t_src = _mm(hp, w_src, b_src)
        w_dst = jnp.concatenate([
            jnp.concatenate([lg["B1_w"], zcol], 0),
            jnp.concatenate([lt["Q_w"], zcol], 0)], 1)
        b_dst = jnp.concatenate([lg["B1_b"], lt["Q_b"]])
        t_dst = _mm(hp, w_dst, b_dst)
        w_a1c1 = jnp.concatenate([
            lg["A1_w"], jnp.concatenate([zcol, lg["C1_w"]], 0)], 1)
        b_a1c1 = jnp.concatenate([lg["A1_b"], lg["C1_b"]])
        t_a1c1 = _mm(hp, w_a1c1, b_a1c1)
        a1h = t_a1c1[:, :d]
        c1p = t_a1c1[:, d:]

        gsrc = _sc_gather(t_src, src_g)
        gdst = _sc_gather(t_dst, dst_g)

        e_hat, sigma, sa, sc_v, e_stats = _edge_gcn(
            e_arr, gdst, gsrc, lg["B3_w"], lg["B3_b"], out_rows=e_pad)

        ssig = _sc_scatter_add(sigma, dst_s, n_acc)
        snh = _sc_scatter_add(sa, dst_s, n_acc)
        snp = _sc_scatter_add(sc_v, dst_s, n_acc)

        h_raw, p_new, h_stats = _node_gcn_a(a1h, c1p, ssig, snh, snp, p, p_i)
        h1 = _bn_apply(h_raw, h, h_stats, lg["bn_h_g"], lg["bn_h_b"],
                       float(n))
        e1 = _bn_apply(e_hat, e_arr, e_stats, lg["bn_e_g"], lg["bn_e_b"],
                       float(e))

        e_att, wb, wv = _gt_edge(e_arr, gdst, gsrc, gsrc,
                                 lt["E_w"], lt["E_b"], out_rows=e_pad)
        swv = _sc_scatter_add(wv, dst_s, n_acc)
        swb = _sc_scatter_add(wb, dst_s, n_acc)

        h = _gt_post(swv, swb, h, lt["Oh_w"], lt["Oh_b"],
                     (lt["ln1_h_g"], lt["ln1_h_b"]),
                     (lt["ffh_w1"], lt["ffh_b1"]),
                     (lt["ffh_w2"], lt["ffh_b2"]),
                     (lt["ln2_h_g"], lt["ln2_h_b"]),
                     h1, h_i, w1v, w2v)
        e_arr = _gt_post(e_att, None, e_arr, lt["Oe_w"], lt["Oe_b"],
                         (lt["ln1_e_g"], lt["ln1_e_b"]),
                         (lt["ffe_w1"], lt["ffe_b1"]),
                         (lt["ffe_w2"], lt["ffe_b2"]),
                         (lt["ln2_e_g"], lt["ln2_e_b"]),
                         e1, e_i, w1v, w2v)
        p = p_new

    n_graphs = 64
    pool_pad_rows = 128
    n_pad = ((n + _NW * 128 - 1) // (_NW * 128)) * (_NW * 128)
    extra = n_pad - n
    h_pool = h if extra == 0 else jnp.concatenate(
        [h, jnp.zeros((extra, d), F32)], axis=0)
    gid = graph_ids.astype(jnp.int32)
    gid_pad = gid if extra == 0 else jnp.concatenate(
        [gid, jnp.full((extra,), n_graphs, jnp.int32)])
    hg_parts = _sc_scatter_add(h_pool, gid_pad, pool_pad_rows)

    return _head(hg_parts, n_graphs, params["mlp_bn_g"], params["mlp_bn_b"],
                 params["mlp_w1"], params["mlp_b1"],
                 params["mlp_w2"], params["mlp_b2"])
